# baseline (speedup 1.0000x reference)
import jax
import jax.numpy as jnp
from jax.experimental import pallas as pl

_IN_DIM = 128
_HID_DIM = 128
_BLK = 2000


def _linear_body(x_ref, w_ref, b_ref, o_ref):
    acc = jax.lax.dot_general(
        x_ref[...],
        w_ref[...],
        dimension_numbers=(((1,), (1,)), ((), ())),
        preferred_element_type=jnp.float32,
    )
    o_ref[...] = acc + b_ref[...]


@jax.jit
def kernel(inputs, W, b):
    n = inputs.shape[0]
    grid = (n // _BLK,)
    return pl.pallas_call(
        _linear_body,
        grid=grid,
        in_specs=[
            pl.BlockSpec((_BLK, _IN_DIM), lambda i: (i, 0)),
            pl.BlockSpec((_HID_DIM, _IN_DIM), lambda i: (0, 0)),
            pl.BlockSpec((1, _HID_DIM), lambda i: (0, 0)),
        ],
        out_specs=pl.BlockSpec((_BLK, _HID_DIM), lambda i: (i, 0)),
        out_shape=jax.ShapeDtypeStruct((n, _HID_DIM), jnp.float32),
    )(inputs, W, b.reshape(1, _HID_DIM))


# speedup vs baseline: 1.3661x; 1.3661x over previous
import jax
import jax.numpy as jnp
from jax.experimental import pallas as pl
from jax.experimental.pallas import tpu as pltpu

_IN_DIM = 128
_HID_DIM = 128
_BLK = 5000


def _linear_body(x_ref, wt_ref, b_ref, o_ref):
    acc = jnp.dot(x_ref[...], wt_ref[...], preferred_element_type=jnp.float32)
    o_ref[...] = acc + b_ref[...]


@jax.jit
def kernel(inputs, W, b):
    n = inputs.shape[0]
    grid = (n // _BLK,)
    return pl.pallas_call(
        _linear_body,
        grid=grid,
        in_specs=[
            pl.BlockSpec((_BLK, _IN_DIM), lambda i: (i, 0)),
            pl.BlockSpec((_IN_DIM, _HID_DIM), lambda i: (0, 0)),
            pl.BlockSpec((1, _HID_DIM), lambda i: (0, 0)),
        ],
        out_specs=pl.BlockSpec((_BLK, _HID_DIM), lambda i: (i, 0)),
        out_shape=jax.ShapeDtypeStruct((n, _HID_DIM), jnp.float32),
        compiler_params=pltpu.CompilerParams(
            dimension_semantics=("parallel",),
        ),
    )(inputs, W.T, b.reshape(1, _HID_DIM))
